# baseline (speedup 1.0000x reference)
import functools

import jax
import jax.numpy as jnp
from jax import lax
from jax.experimental import pallas as pl
from jax.experimental.pallas import tpu as pltpu
from jax.experimental.pallas import tpu_sc as plsc

_C = 8
_R = 4


def kernel(S, pe_weight):
    B, T, D = S.shape
    info = plsc.get_sparse_core_info()
    nw = info.num_cores * info.num_subcores
    pos_per_w = T // nw
    n_iter = pos_per_w // _C
    mesh = plsc.VectorSubcoreMesh(core_axis_name="c", subcore_axis_name="s")

    @functools.partial(
        pl.kernel,
        out_type=jax.ShapeDtypeStruct((B, T, D), jnp.float32),
        mesh=mesh,
        scratch_types=[
            pltpu.VMEM((2, _C, D), jnp.float32),
            pltpu.VMEM((_R, B, _C, D), jnp.float32),
            pltpu.SemaphoreType.DMA((_R,)),
            pltpu.SemaphoreType.DMA((_R,)),
            pltpu.SemaphoreType.DMA((2,)),
        ],
    )
    def sc_add(s_hbm, pe_hbm, out_hbm, pe_buf, s_buf, in_sem, out_sem, pe_sem):
        wid = lax.axis_index("s") * info.num_cores + lax.axis_index("c")
        base = wid * pos_per_w

        def start_in(j):
            return pltpu.async_copy(
                s_hbm.at[:, pl.ds(base + j * _C, _C), :],
                s_buf.at[j % _R], in_sem.at[j % _R])

        def start_pe(j):
            return pltpu.async_copy(
                pe_hbm.at[pl.ds(base + j * _C, _C), :],
                pe_buf.at[j % 2], pe_sem.at[j % 2])

        for j0 in range(_R - 1):
            start_in(j0)
        start_pe(0)
        start_pe(1)

        def body(j, carry):
            slot = j % _R
            pj = j % 2

            pltpu.make_async_copy(
                s_hbm.at[:, pl.ds(base + j * _C, _C), :],
                s_buf.at[slot], in_sem.at[slot]).wait()
            pltpu.make_async_copy(
                pe_hbm.at[pl.ds(base + j * _C, _C), :],
                pe_buf.at[pj], pe_sem.at[pj]).wait()

            @plsc.parallel_loop(0, _C, unroll=8)
            def _(r):
                for g in range(D // 16):
                    v = pe_buf[pj, r, pl.ds(g * 16, 16)]
                    for b in range(B):
                        plsc.addupdate(
                            s_buf.at[slot, b, r, pl.ds(g * 16, 16)], v)

            pltpu.async_copy(
                s_buf.at[slot], out_hbm.at[:, pl.ds(base + j * _C, _C), :],
                out_sem.at[slot])

            @pl.when(j + 2 < n_iter)
            def _():
                start_pe(j + 2)

            @pl.when(j >= 1)
            def _():
                jm = j - 1
                pltpu.make_async_copy(
                    s_buf.at[jm % _R],
                    out_hbm.at[:, pl.ds(base + jm * _C, _C), :],
                    out_sem.at[jm % _R]).wait()

            @pl.when(j + _R - 1 < n_iter)
            def _():
                start_in(j + _R - 1)

            return carry

        lax.fori_loop(0, n_iter, body, 0)

        j = n_iter - 1
        pltpu.make_async_copy(
            s_buf.at[j % _R], out_hbm.at[:, pl.ds(base + j * _C, _C), :],
            out_sem.at[j % _R]).wait()

    return sc_add(S, pe_weight)


# speedup vs baseline: 1.1725x; 1.1725x over previous
import functools

import jax
import jax.numpy as jnp
from jax import lax
from jax.experimental import pallas as pl
from jax.experimental.pallas import tpu as pltpu
from jax.experimental.pallas import tpu_sc as plsc

_C = 8
_R = 4


def kernel(S, pe_weight):
    B, T, D = S.shape
    info = plsc.get_sparse_core_info()
    nw = info.num_cores * info.num_subcores
    pos_per_w = T // nw
    n_iter = pos_per_w // _C
    mesh = plsc.VectorSubcoreMesh(core_axis_name="c", subcore_axis_name="s")

    @functools.partial(
        pl.kernel,
        out_type=jax.ShapeDtypeStruct((B, T, D), jnp.float32),
        mesh=mesh,
        scratch_types=[
            pltpu.VMEM((2, _C, D), jnp.float32),
            pltpu.VMEM((_R, B, _C, D), jnp.float32),
            pltpu.SemaphoreType.DMA((_R,)),
            pltpu.SemaphoreType.DMA((_R,)),
            pltpu.SemaphoreType.DMA((2,)),
        ],
    )
    def sc_add(s_hbm, pe_hbm, out_hbm, pe_buf, s_buf, in_sem, out_sem, pe_sem):
        wid = lax.axis_index("s") * info.num_cores + lax.axis_index("c")
        base = wid * pos_per_w

        def start_in(j):
            return pltpu.async_copy(
                s_hbm.at[:, pl.ds(base + j * _C, _C), :],
                s_buf.at[j % _R], in_sem.at[j % _R])

        def start_pe(j):
            return pltpu.async_copy(
                pe_hbm.at[pl.ds(base + j * _C, _C), :],
                pe_buf.at[j % 2], pe_sem.at[j % 2])

        for j0 in range(_R - 1):
            start_in(j0)
        start_pe(0)
        start_pe(1)

        def body(j, carry):
            slot = j % _R
            pj = j % 2

            pltpu.make_async_copy(
                s_hbm.at[:, pl.ds(base + j * _C, _C), :],
                s_buf.at[slot], in_sem.at[slot]).wait()
            pltpu.make_async_copy(
                pe_hbm.at[pl.ds(base + j * _C, _C), :],
                pe_buf.at[pj], pe_sem.at[pj]).wait()

            @plsc.parallel_loop(0, _C, unroll=2)
            def _(r):
                for g in range(D // 16):
                    v = pe_buf[pj, r, pl.ds(g * 16, 16)]
                    for b in range(B):
                        plsc.addupdate(
                            s_buf.at[slot, b, r, pl.ds(g * 16, 16)], v)

            pltpu.async_copy(
                s_buf.at[slot], out_hbm.at[:, pl.ds(base + j * _C, _C), :],
                out_sem.at[slot])

            @pl.when(j + 2 < n_iter)
            def _():
                start_pe(j + 2)

            @pl.when(j >= 1)
            def _():
                jm = j - 1
                pltpu.make_async_copy(
                    s_buf.at[jm % _R],
                    out_hbm.at[:, pl.ds(base + jm * _C, _C), :],
                    out_sem.at[jm % _R]).wait()

            @pl.when(j + _R - 1 < n_iter)
            def _():
                start_in(j + _R - 1)

            return carry

        lax.fori_loop(0, n_iter, body, 0)

        j = n_iter - 1
        pltpu.make_async_copy(
            s_buf.at[j % _R], out_hbm.at[:, pl.ds(base + j * _C, _C), :],
            out_sem.at[j % _R]).wait()

    return sc_add(S, pe_weight)


# speedup vs baseline: 1.1790x; 1.0055x over previous
import functools

import jax
import jax.numpy as jnp
from jax import lax
from jax.experimental import pallas as pl
from jax.experimental.pallas import tpu as pltpu
from jax.experimental.pallas import tpu_sc as plsc

_C = 16
_R = 6


def kernel(S, pe_weight):
    B, T, D = S.shape
    info = plsc.get_sparse_core_info()
    nw = info.num_cores * info.num_subcores
    pos_per_w = T // nw
    n_chunks = pos_per_w // _C
    n_iter = n_chunks * B
    mesh = plsc.VectorSubcoreMesh(core_axis_name="c", subcore_axis_name="s")

    @functools.partial(
        pl.kernel,
        out_type=jax.ShapeDtypeStruct((B, T, D), jnp.float32),
        mesh=mesh,
        scratch_types=[
            pltpu.VMEM((2, _C, D), jnp.float32),
            pltpu.VMEM((_R, _C, D), jnp.float32),
            pltpu.SemaphoreType.DMA((_R,)),
            pltpu.SemaphoreType.DMA((_R,)),
            pltpu.SemaphoreType.DMA((2,)),
        ],
    )
    def sc_add(s_hbm, pe_hbm, out_hbm, pe_buf, s_buf, in_sem, out_sem, pe_sem):
        wid = lax.axis_index("s") * info.num_cores + lax.axis_index("c")
        base = wid * pos_per_w

        def start_in(j, slot):
            k = j // B
            b = j % B
            return pltpu.async_copy(
                s_hbm.at[b, pl.ds(base + k * _C, _C), :],
                s_buf.at[slot], in_sem.at[slot])

        def start_pe(k):
            return pltpu.async_copy(
                pe_hbm.at[pl.ds(base + k * _C, _C), :],
                pe_buf.at[k % 2], pe_sem.at[k % 2])

        for j0 in range(_R - 1):
            start_in(j0, j0)
        start_pe(0)

        def body(j, carry):
            k = j // B
            b = j % B
            slot = j % _R
            kp = k % 2

            pltpu.make_async_copy(
                s_hbm.at[b, pl.ds(base + k * _C, _C), :],
                s_buf.at[slot], in_sem.at[slot]).wait()

            @pl.when(b == 0)
            def _():
                pltpu.make_async_copy(
                    pe_hbm.at[pl.ds(base + k * _C, _C), :],
                    pe_buf.at[kp], pe_sem.at[kp]).wait()

            @pl.when(jnp.logical_and(b == 1, k + 1 < n_chunks))
            def _():
                start_pe(k + 1)

            @plsc.parallel_loop(0, _C, unroll=2)
            def _(r):
                for g in range(D // 16):
                    v = pe_buf[kp, r, pl.ds(g * 16, 16)]
                    plsc.addupdate(s_buf.at[slot, r, pl.ds(g * 16, 16)], v)

            pltpu.async_copy(
                s_buf.at[slot], out_hbm.at[b, pl.ds(base + k * _C, _C), :],
                out_sem.at[slot])

            @pl.when(j >= 1)
            def _():
                jm = j - 1
                km = jm // B
                bm = jm % B
                pltpu.make_async_copy(
                    s_buf.at[jm % _R],
                    out_hbm.at[bm, pl.ds(base + km * _C, _C), :],
                    out_sem.at[jm % _R]).wait()

            @pl.when(j + _R - 1 < n_iter)
            def _():
                start_in(j + _R - 1, (j + _R - 1) % _R)

            return carry

        lax.fori_loop(0, n_iter, body, 0)

        j = n_iter - 1
        k, b = j // B, j % B
        pltpu.make_async_copy(
            s_buf.at[j % _R], out_hbm.at[b, pl.ds(base + k * _C, _C), :],
            out_sem.at[j % _R]).wait()

    return sc_add(S, pe_weight)
